# Initial kernel scaffold; baseline (speedup 1.0000x reference)
import jax
import jax.numpy as jnp
from jax.experimental import pallas as pl


def kernel(x, resolution, origin):
    raise NotImplementedError("write your pallas kernel here")


# speedup vs baseline: 1.5511x; 1.5511x over previous
import jax
import jax.numpy as jnp
from jax import lax
from jax.experimental import pallas as pl
from jax.experimental.pallas import tpu as pltpu
from jax.experimental.pallas import tpu_sc as plsc

_B = 16
_SEQ = 20
_N = 32
_NP = _N // 2
_H = 64
_W = 64
_BT = _B * _SEQ
_IMG = _H * _W * _NP
_TOTAL = _BT * _IMG
_PTS = _BT * _NP

_NC = 2
_NS = 16
_NW = _NC * _NS
_IMGS_PER_TILE = _BT // _NW


def _sc_body(xs_hbm, ys_hbm, rx_hbm, ry_hbm, ox_hbm, oy_hbm, out_hbm,
             zbuf, xsv, ysv, rxv, ryv, oxv, oyv, onesv,
             fill_sem, scat_sem):
    wid = lax.axis_index("s") * _NC + lax.axis_index("c")
    base = wid * _IMGS_PER_TILE

    pltpu.sync_copy(xs_hbm, xsv)
    pltpu.sync_copy(ys_hbm, ysv)
    pltpu.sync_copy(rx_hbm, rxv)
    pltpu.sync_copy(ry_hbm, ryv)
    pltpu.sync_copy(ox_hbm, oxv)
    pltpu.sync_copy(oy_hbm, oyv)
    onesv[...] = jnp.ones((16,), jnp.float32)

    zeros16 = jnp.zeros((16,), jnp.float32)

    def _zero(i, carry):
        zbuf[pl.ds(i * 16, 16)] = zeros16
        return carry

    lax.fori_loop(0, _IMG // 16, _zero, 0, unroll=8)

    for i in range(_IMGS_PER_TILE):
        bt = base + i
        pltpu.async_copy(zbuf, out_hbm.at[pl.ds(bt * _IMG, _IMG)], fill_sem)
    for i in range(_IMGS_PER_TILE):
        bt = base + i
        pltpu.make_async_copy(
            zbuf, out_hbm.at[pl.ds(bt * _IMG, _IMG)], fill_sem).wait()

    lane = lax.iota(jnp.int32, 16)
    for i in range(_IMGS_PER_TILE):
        bt = base + i
        s = bt * _NP
        xs = xsv[pl.ds(s, 16)]
        ys = ysv[pl.ds(s, 16)]
        rx = rxv[pl.ds(s, 16)]
        ry = ryv[pl.ds(s, 16)]
        ox = oxv[pl.ds(s, 16)]
        oy = oyv[pl.ds(s, 16)]
        col = (xs / rx + ox).astype(jnp.int32)
        row = (ys / ry + oy).astype(jnp.int32)
        offs = bt * _IMG + row * (_W * _NP) + col * _NP + lane
        pltpu.async_copy(onesv, out_hbm.at[offs], scat_sem).wait()


@jax.jit
def _sc_raster(xs, ys, rx, ry, ox, oy):
    mesh = plsc.VectorSubcoreMesh(core_axis_name="c", subcore_axis_name="s")
    return pl.kernel(
        _sc_body,
        out_type=jax.ShapeDtypeStruct((_TOTAL,), jnp.float32),
        mesh=mesh,
        scratch_types=[
            pltpu.VMEM((_IMG,), jnp.float32),
            pltpu.VMEM((_PTS,), jnp.float32),
            pltpu.VMEM((_PTS,), jnp.float32),
            pltpu.VMEM((_PTS,), jnp.float32),
            pltpu.VMEM((_PTS,), jnp.float32),
            pltpu.VMEM((_PTS,), jnp.float32),
            pltpu.VMEM((_PTS,), jnp.float32),
            pltpu.VMEM((16,), jnp.float32),
            pltpu.SemaphoreType.DMA,
            pltpu.SemaphoreType.DMA,
        ],
    )(xs, ys, rx, ry, ox, oy)


def kernel(x, resolution, origin):
    pts = x.reshape(_PTS, 2)
    xs = pts[:, 0]
    ys = pts[:, 1]
    rx = jnp.broadcast_to(resolution[:, :, None, 0], (_B, _SEQ, _NP)).reshape(-1)
    ry = jnp.broadcast_to(resolution[:, :, None, 1], (_B, _SEQ, _NP)).reshape(-1)
    ox = jnp.broadcast_to(origin[:, :, None, 0], (_B, _SEQ, _NP)).reshape(-1)
    oy = jnp.broadcast_to(origin[:, :, None, 1], (_B, _SEQ, _NP)).reshape(-1)
    out = _sc_raster(xs, ys, rx, ry, ox, oy)
    return out.reshape(_B, _SEQ, _H, _W, _NP)


# speedup vs baseline: 4.5133x; 2.9096x over previous
import jax
import jax.numpy as jnp
from jax import lax
from jax.experimental import pallas as pl
from jax.experimental.pallas import tpu as pltpu
from jax.experimental.pallas import tpu_sc as plsc

_B = 16
_SEQ = 20
_N = 32
_NP = _N // 2
_H = 64
_W = 64
_BT = _B * _SEQ
_IMG = _H * _W * _NP
_TOTAL = _BT * _IMG
_PTS = _BT * _NP

_NC = 2
_NS = 16
_NW = _NC * _NS
_IMGS_PER_TILE = _BT // _NW


def _sc_body(xs_hbm, ys_hbm, rx_hbm, ry_hbm, ox_hbm, oy_hbm, out_hbm,
             zbuf, xsv, ysv, rxv, ryv, oxv, oyv, onesv,
             fill_sem, scat_sem):
    wid = lax.axis_index("s") * _NC + lax.axis_index("c")
    base = wid * _IMGS_PER_TILE

    pltpu.sync_copy(xs_hbm, xsv)
    pltpu.sync_copy(ys_hbm, ysv)
    pltpu.sync_copy(rx_hbm, rxv)
    pltpu.sync_copy(ry_hbm, ryv)
    pltpu.sync_copy(ox_hbm, oxv)
    pltpu.sync_copy(oy_hbm, oyv)
    onesv[...] = jnp.ones((16,), jnp.float32)

    zeros16 = jnp.zeros((16,), jnp.float32)

    def _zero(i, carry):
        zbuf[pl.ds(i * 16, 16)] = zeros16
        return carry

    lax.fori_loop(0, _IMG // 16, _zero, 0, unroll=8)

    for i in range(_IMGS_PER_TILE):
        bt = base + i
        pltpu.async_copy(zbuf, out_hbm.at[pl.ds(bt * _IMG, _IMG)], fill_sem)
    for i in range(_IMGS_PER_TILE):
        bt = base + i
        pltpu.make_async_copy(
            zbuf, out_hbm.at[pl.ds(bt * _IMG, _IMG)], fill_sem).wait()

    lane = lax.iota(jnp.int32, 16)
    for i in range(_IMGS_PER_TILE):
        bt = base + i
        s = bt * _NP
        xs = xsv[pl.ds(s, 16)]
        ys = ysv[pl.ds(s, 16)]
        rx = rxv[pl.ds(s, 16)]
        ry = ryv[pl.ds(s, 16)]
        ox = oxv[pl.ds(s, 16)]
        oy = oyv[pl.ds(s, 16)]
        col = (xs / rx + ox).astype(jnp.int32)
        row = (ys / ry + oy).astype(jnp.int32)
        offs = bt * _IMG + row * (_NP * _W) + lane * _W + col
        pltpu.async_copy(onesv, out_hbm.at[offs], scat_sem).wait()


@jax.jit
def _sc_raster(xs, ys, rx, ry, ox, oy):
    mesh = plsc.VectorSubcoreMesh(core_axis_name="c", subcore_axis_name="s")
    return pl.kernel(
        _sc_body,
        out_type=jax.ShapeDtypeStruct((_TOTAL,), jnp.float32),
        mesh=mesh,
        scratch_types=[
            pltpu.VMEM((_IMG,), jnp.float32),
            pltpu.VMEM((_PTS,), jnp.float32),
            pltpu.VMEM((_PTS,), jnp.float32),
            pltpu.VMEM((_PTS,), jnp.float32),
            pltpu.VMEM((_PTS,), jnp.float32),
            pltpu.VMEM((_PTS,), jnp.float32),
            pltpu.VMEM((_PTS,), jnp.float32),
            pltpu.VMEM((16,), jnp.float32),
            pltpu.SemaphoreType.DMA,
            pltpu.SemaphoreType.DMA,
        ],
    )(xs, ys, rx, ry, ox, oy)


def kernel(x, resolution, origin):
    pts = x.reshape(_PTS, 2)
    xs = pts[:, 0]
    ys = pts[:, 1]
    rx = jnp.broadcast_to(resolution[:, :, None, 0], (_B, _SEQ, _NP)).reshape(-1)
    ry = jnp.broadcast_to(resolution[:, :, None, 1], (_B, _SEQ, _NP)).reshape(-1)
    ox = jnp.broadcast_to(origin[:, :, None, 0], (_B, _SEQ, _NP)).reshape(-1)
    oy = jnp.broadcast_to(origin[:, :, None, 1], (_B, _SEQ, _NP)).reshape(-1)
    out = _sc_raster(xs, ys, rx, ry, ox, oy)
    return jnp.swapaxes(out.reshape(_B, _SEQ, _H, _NP, _W), 3, 4)


# speedup vs baseline: 7.9106x; 1.7527x over previous
import jax
import jax.numpy as jnp
from jax import lax
from jax.experimental import pallas as pl
from jax.experimental.pallas import tpu as pltpu
from jax.experimental.pallas import tpu_sc as plsc

_B = 16
_SEQ = 20
_N = 32
_NP = _N // 2
_H = 64
_W = 64
_BT = _B * _SEQ
_PTS = _BT * _NP

_NC = 2
_NS = 16
_NW = _NC * _NS
_IMGS_PER_TILE = _BT // _NW
_PPT = _IMGS_PER_TILE * _NP


def _sc_body(xs_hbm, ys_hbm, rx_hbm, ry_hbm, ox_hbm, oy_hbm, idx_hbm,
             xsv, ysv, rxv, ryv, oxv, oyv, idxv):
    wid = lax.axis_index("s") * _NC + lax.axis_index("c")
    base = wid * _PPT

    pltpu.sync_copy(xs_hbm.at[pl.ds(base, _PPT)], xsv)
    pltpu.sync_copy(ys_hbm.at[pl.ds(base, _PPT)], ysv)
    pltpu.sync_copy(rx_hbm.at[pl.ds(base, _PPT)], rxv)
    pltpu.sync_copy(ry_hbm.at[pl.ds(base, _PPT)], ryv)
    pltpu.sync_copy(ox_hbm.at[pl.ds(base, _PPT)], oxv)
    pltpu.sync_copy(oy_hbm.at[pl.ds(base, _PPT)], oyv)

    for i in range(_IMGS_PER_TILE):
        s = i * _NP
        xs = xsv[pl.ds(s, 16)]
        ys = ysv[pl.ds(s, 16)]
        rx = rxv[pl.ds(s, 16)]
        ry = ryv[pl.ds(s, 16)]
        ox = oxv[pl.ds(s, 16)]
        oy = oyv[pl.ds(s, 16)]
        col = (xs / rx + ox).astype(jnp.int32)
        row = (ys / ry + oy).astype(jnp.int32)
        idxv[pl.ds(s, 16)] = row * _W + col

    pltpu.sync_copy(idxv, idx_hbm.at[pl.ds(base, _PPT)])


@jax.jit
def _sc_indices(xs, ys, rx, ry, ox, oy):
    mesh = plsc.VectorSubcoreMesh(core_axis_name="c", subcore_axis_name="s")
    return pl.kernel(
        _sc_body,
        out_type=jax.ShapeDtypeStruct((_PTS,), jnp.int32),
        mesh=mesh,
        scratch_types=[
            pltpu.VMEM((_PPT,), jnp.float32),
            pltpu.VMEM((_PPT,), jnp.float32),
            pltpu.VMEM((_PPT,), jnp.float32),
            pltpu.VMEM((_PPT,), jnp.float32),
            pltpu.VMEM((_PPT,), jnp.float32),
            pltpu.VMEM((_PPT,), jnp.float32),
            pltpu.VMEM((_PPT,), jnp.int32),
        ],
    )(xs, ys, rx, ry, ox, oy)


_TPB = 4
_NBLK = _BT // _TPB


def _tc_body(idx_ref, out_ref):
    out_ref[...] = jnp.zeros((1, _TPB, _H, _NP, _W), jnp.float32)
    wio = lax.broadcasted_iota(jnp.int32, (1, 1, _W), 2)
    for tl in range(_TPB):
        for p in range(_NP):
            v = idx_ref[0, tl, p]
            r = v // _W
            c = v - r * _W
            oh = jnp.where(wio == c, 1.0, 0.0).astype(jnp.float32)
            out_ref[0, tl, pl.ds(r, 1), pl.ds(p, 1), :] = oh


@jax.jit
def _tc_raster(idx3):
    return pl.pallas_call(
        _tc_body,
        grid=(_NBLK,),
        in_specs=[pl.BlockSpec((1, _TPB, _NP), lambda i: (i, 0, 0),
                               memory_space=pltpu.SMEM)],
        out_specs=pl.BlockSpec((1, _TPB, _H, _NP, _W),
                               lambda i: (i // (_SEQ // _TPB),
                                          i % (_SEQ // _TPB), 0, 0, 0)),
        out_shape=jax.ShapeDtypeStruct((_B, _SEQ, _H, _NP, _W), jnp.float32),
    )(idx3)


def kernel(x, resolution, origin):
    pts = x.reshape(_PTS, 2)
    xs = pts[:, 0]
    ys = pts[:, 1]
    rx = jnp.broadcast_to(resolution[:, :, None, 0], (_B, _SEQ, _NP)).reshape(-1)
    ry = jnp.broadcast_to(resolution[:, :, None, 1], (_B, _SEQ, _NP)).reshape(-1)
    ox = jnp.broadcast_to(origin[:, :, None, 0], (_B, _SEQ, _NP)).reshape(-1)
    oy = jnp.broadcast_to(origin[:, :, None, 1], (_B, _SEQ, _NP)).reshape(-1)
    idx = _sc_indices(xs, ys, rx, ry, ox, oy)
    out = _tc_raster(idx.reshape(_NBLK, _TPB, _NP))
    return jnp.swapaxes(out, 3, 4)


# speedup vs baseline: 9.7147x; 1.2281x over previous
import jax
import jax.numpy as jnp
from jax import lax
from jax.experimental import pallas as pl
from jax.experimental.pallas import tpu as pltpu
from jax.experimental.pallas import tpu_sc as plsc

_B = 16
_SEQ = 20
_N = 32
_NP = _N // 2
_H = 64
_W = 64
_BT = _B * _SEQ
_PTS = _BT * _NP

_NC = 2
_NS = 16
_NW = _NC * _NS
_IMGS_PER_TILE = _BT // _NW
_PPT = _IMGS_PER_TILE * _NP


def _sc_body(xs_hbm, ys_hbm, rx_hbm, ry_hbm, ox_hbm, oy_hbm, idx_hbm,
             xsv, ysv, rxv, ryv, oxv, oyv, idxv):
    wid = lax.axis_index("s") * _NC + lax.axis_index("c")
    base = wid * _PPT

    pltpu.sync_copy(xs_hbm.at[pl.ds(base, _PPT)], xsv)
    pltpu.sync_copy(ys_hbm.at[pl.ds(base, _PPT)], ysv)
    pltpu.sync_copy(rx_hbm.at[pl.ds(base, _PPT)], rxv)
    pltpu.sync_copy(ry_hbm.at[pl.ds(base, _PPT)], ryv)
    pltpu.sync_copy(ox_hbm.at[pl.ds(base, _PPT)], oxv)
    pltpu.sync_copy(oy_hbm.at[pl.ds(base, _PPT)], oyv)

    for i in range(_IMGS_PER_TILE):
        s = i * _NP
        xs = xsv[pl.ds(s, 16)]
        ys = ysv[pl.ds(s, 16)]
        rx = rxv[pl.ds(s, 16)]
        ry = ryv[pl.ds(s, 16)]
        ox = oxv[pl.ds(s, 16)]
        oy = oyv[pl.ds(s, 16)]
        col = (xs / rx + ox).astype(jnp.int32)
        row = (ys / ry + oy).astype(jnp.int32)
        idxv[pl.ds(s, 16)] = row * _W + col

    pltpu.sync_copy(idxv, idx_hbm.at[pl.ds(base, _PPT)])


@jax.jit
def _sc_indices(xs, ys, rx, ry, ox, oy):
    mesh = plsc.VectorSubcoreMesh(core_axis_name="c", subcore_axis_name="s")
    return pl.kernel(
        _sc_body,
        out_type=jax.ShapeDtypeStruct((_PTS,), jnp.int32),
        mesh=mesh,
        scratch_types=[
            pltpu.VMEM((_PPT,), jnp.float32),
            pltpu.VMEM((_PPT,), jnp.float32),
            pltpu.VMEM((_PPT,), jnp.float32),
            pltpu.VMEM((_PPT,), jnp.float32),
            pltpu.VMEM((_PPT,), jnp.float32),
            pltpu.VMEM((_PPT,), jnp.float32),
            pltpu.VMEM((_PPT,), jnp.int32),
        ],
    )(xs, ys, rx, ry, ox, oy)


_TPB = 10
_NBLK = _BT // _TPB


def _tc_body(idx_ref, out_ref):
    out_ref[...] = jnp.zeros((1, _TPB, _H, _NP, _W), jnp.float32)
    wio = lax.broadcasted_iota(jnp.int32, (1, 1, _W), 2)
    for tl in range(_TPB):
        for p in range(_NP):
            v = idx_ref[0, tl, p]
            r = v // _W
            c = v - r * _W
            oh = jnp.where(wio == c, 1.0, 0.0).astype(jnp.float32)
            out_ref[0, tl, pl.ds(r, 1), pl.ds(p, 1), :] = oh


@jax.jit
def _tc_raster(idx3):
    return pl.pallas_call(
        _tc_body,
        grid=(_NBLK,),
        in_specs=[pl.BlockSpec((1, _TPB, _NP), lambda i: (i, 0, 0),
                               memory_space=pltpu.SMEM)],
        out_specs=pl.BlockSpec((1, _TPB, _H, _NP, _W),
                               lambda i: (i // (_SEQ // _TPB),
                                          i % (_SEQ // _TPB), 0, 0, 0)),
        out_shape=jax.ShapeDtypeStruct((_B, _SEQ, _H, _NP, _W), jnp.float32),
    )(idx3)


def kernel(x, resolution, origin):
    pts = x.reshape(_PTS, 2)
    xs = pts[:, 0]
    ys = pts[:, 1]
    rx = jnp.broadcast_to(resolution[:, :, None, 0], (_B, _SEQ, _NP)).reshape(-1)
    ry = jnp.broadcast_to(resolution[:, :, None, 1], (_B, _SEQ, _NP)).reshape(-1)
    ox = jnp.broadcast_to(origin[:, :, None, 0], (_B, _SEQ, _NP)).reshape(-1)
    oy = jnp.broadcast_to(origin[:, :, None, 1], (_B, _SEQ, _NP)).reshape(-1)
    idx = _sc_indices(xs, ys, rx, ry, ox, oy)
    out = _tc_raster(idx.reshape(_NBLK, _TPB, _NP))
    return jnp.swapaxes(out, 3, 4)


# speedup vs baseline: 10.1014x; 1.0398x over previous
import jax
import jax.numpy as jnp
from jax import lax
from jax.experimental import pallas as pl
from jax.experimental.pallas import tpu as pltpu
from jax.experimental.pallas import tpu_sc as plsc

_B = 16
_SEQ = 20
_N = 32
_NP = _N // 2
_H = 64
_W = 64
_BT = _B * _SEQ
_PTS = _BT * _NP

_NC = 2
_NS = 16
_NW = _NC * _NS
_IMGS_PER_TILE = _BT // _NW
_PPT = _IMGS_PER_TILE * _NP


def _sc_body(xs_hbm, ys_hbm, rx_hbm, ry_hbm, ox_hbm, oy_hbm, idx_hbm,
             xsv, ysv, rxv, ryv, oxv, oyv, idxv):
    wid = lax.axis_index("s") * _NC + lax.axis_index("c")
    base = wid * _PPT

    pltpu.sync_copy(xs_hbm.at[pl.ds(base, _PPT)], xsv)
    pltpu.sync_copy(ys_hbm.at[pl.ds(base, _PPT)], ysv)
    pltpu.sync_copy(rx_hbm.at[pl.ds(base, _PPT)], rxv)
    pltpu.sync_copy(ry_hbm.at[pl.ds(base, _PPT)], ryv)
    pltpu.sync_copy(ox_hbm.at[pl.ds(base, _PPT)], oxv)
    pltpu.sync_copy(oy_hbm.at[pl.ds(base, _PPT)], oyv)

    for i in range(_IMGS_PER_TILE):
        s = i * _NP
        xs = xsv[pl.ds(s, 16)]
        ys = ysv[pl.ds(s, 16)]
        rx = rxv[pl.ds(s, 16)]
        ry = ryv[pl.ds(s, 16)]
        ox = oxv[pl.ds(s, 16)]
        oy = oyv[pl.ds(s, 16)]
        col = (xs / rx + ox).astype(jnp.int32)
        row = (ys / ry + oy).astype(jnp.int32)
        idxv[pl.ds(s, 16)] = row * _W + col

    pltpu.sync_copy(idxv, idx_hbm.at[pl.ds(base, _PPT)])


@jax.jit
def _sc_indices(xs, ys, rx, ry, ox, oy):
    mesh = plsc.VectorSubcoreMesh(core_axis_name="c", subcore_axis_name="s")
    return pl.kernel(
        _sc_body,
        out_type=jax.ShapeDtypeStruct((_PTS,), jnp.int32),
        mesh=mesh,
        scratch_types=[
            pltpu.VMEM((_PPT,), jnp.float32),
            pltpu.VMEM((_PPT,), jnp.float32),
            pltpu.VMEM((_PPT,), jnp.float32),
            pltpu.VMEM((_PPT,), jnp.float32),
            pltpu.VMEM((_PPT,), jnp.float32),
            pltpu.VMEM((_PPT,), jnp.float32),
            pltpu.VMEM((_PPT,), jnp.int32),
        ],
    )(xs, ys, rx, ry, ox, oy)


_TPB = 20
_NBLK = _BT // _TPB


def _tc_body(idx_ref, out_ref):
    out_ref[...] = jnp.zeros((1, _TPB, _H, _NP, _W), jnp.float32)
    wio = lax.broadcasted_iota(jnp.int32, (1, 1, _W), 2)
    for tl in range(_TPB):
        for p in range(_NP):
            v = idx_ref[0, tl, p]
            r = v // _W
            c = v - r * _W
            oh = jnp.where(wio == c, 1.0, 0.0).astype(jnp.float32)
            out_ref[0, tl, pl.ds(r, 1), pl.ds(p, 1), :] = oh


@jax.jit
def _tc_raster(idx3):
    return pl.pallas_call(
        _tc_body,
        grid=(_NBLK,),
        in_specs=[pl.BlockSpec((1, _TPB, _NP), lambda i: (i, 0, 0),
                               memory_space=pltpu.SMEM)],
        out_specs=pl.BlockSpec((1, _TPB, _H, _NP, _W),
                               lambda i: (i // (_SEQ // _TPB),
                                          i % (_SEQ // _TPB), 0, 0, 0)),
        out_shape=jax.ShapeDtypeStruct((_B, _SEQ, _H, _NP, _W), jnp.float32),
    )(idx3)


def kernel(x, resolution, origin):
    pts = x.reshape(_PTS, 2)
    xs = pts[:, 0]
    ys = pts[:, 1]
    rx = jnp.broadcast_to(resolution[:, :, None, 0], (_B, _SEQ, _NP)).reshape(-1)
    ry = jnp.broadcast_to(resolution[:, :, None, 1], (_B, _SEQ, _NP)).reshape(-1)
    ox = jnp.broadcast_to(origin[:, :, None, 0], (_B, _SEQ, _NP)).reshape(-1)
    oy = jnp.broadcast_to(origin[:, :, None, 1], (_B, _SEQ, _NP)).reshape(-1)
    idx = _sc_indices(xs, ys, rx, ry, ox, oy)
    out = _tc_raster(idx.reshape(_NBLK, _TPB, _NP))
    return jnp.swapaxes(out, 3, 4)
